# baseline (device time: 24266 ns/iter reference)
import jax
import jax.numpy as jnp
from jax import lax
from jax.experimental import pallas as pl
from jax.experimental.pallas import tpu as pltpu

N_DEV = 8
LOG_N = 3
BLK = 64
N_CHUNKS = 4


def kernel(x, Wq, K_ext, V_ext, Wo):
    B, Sq, Dm = x.shape
    _, Dq = Wq.shape
    _, Skv, Hq, Dh = K_ext.shape
    HL = Dq // Dh
    rows_total = B * Sq
    crows = rows_total // N_CHUNKS
    chunks_per_b = N_CHUNKS // B

    xb = x.reshape(rows_total, Dm)

    def body(x_ref, wq_ref, k_any, v_any, wo_ref, out_ref,
             ctx_ref, acc_ref, cbuf_ref, k_ref, v_ref,
             send_sems, recv_sems, copy_sems):
        my_pos = lax.axis_index("i")
        xors = (1, 3, 4)

        heads = pl.ds(my_pos * HL, HL)
        k_cp = pltpu.make_async_copy(
            k_any.at[:, :, heads, :], k_ref, copy_sems.at[0])
        v_cp = pltpu.make_async_copy(
            v_any.at[:, :, heads, :], v_ref, copy_sems.at[1])
        k_cp.start()
        v_cp.start()

        barrier = pltpu.get_barrier_semaphore()
        for r in range(LOG_N):
            pl.semaphore_signal(
                barrier, inc=1,
                device_id=(my_pos ^ xors[r],),
                device_id_type=pl.DeviceIdType.MESH,
            )
        pl.semaphore_wait(barrier, LOG_N)

        i_idx = lax.broadcasted_iota(jnp.int32, (Sq, Skv), 0)
        j_idx = lax.broadcasted_iota(jnp.int32, (Sq, Skv), 1)
        bias = jnp.where((j_idx // BLK) <= (i_idx // BLK), 0.0, -1e9)

        def make_rdma(c, r):
            rows = pl.ds(c * crows, crows)
            return pltpu.make_async_remote_copy(
                src_ref=acc_ref.at[rows],
                dst_ref=cbuf_ref.at[r, rows],
                send_sem=send_sems.at[c, r],
                recv_sem=recv_sems.at[c, r],
                device_id=(my_pos ^ xors[r],),
                device_id_type=pl.DeviceIdType.MESH,
            )

        rdmas = {}
        wqb = wq_ref[...].astype(jnp.bfloat16)
        wob = wo_ref[...].astype(jnp.bfloat16)
        for b in range(B):
            brows = slice(b * Sq, (b + 1) * Sq)
            q = (lax.dot_general(
                x_ref[brows, :].astype(jnp.bfloat16), wqb,
                (((1,), (0,)), ((), ())),
                preferred_element_type=jnp.float32,
            ) * 0.125).astype(jnp.bfloat16)
            if b == 0:
                k_cp.wait()
                v_cp.wait()
            for h in range(HL):
                q_bh = q[:, h * Dh:(h + 1) * Dh]
                k_bh = k_ref[b, :, h, :].astype(jnp.bfloat16)
                v_bh = v_ref[b, :, h, :].astype(jnp.bfloat16)
                s = lax.dot_general(
                    q_bh, k_bh, (((1,), (1,)), ((), ())),
                    preferred_element_type=jnp.float32,
                ) + bias
                w = jnp.exp(s)
                recip = 1.0 / jnp.sum(w, axis=-1, keepdims=True)
                ctx = lax.dot_general(
                    w.astype(jnp.bfloat16), v_bh, (((1,), (0,)), ((), ())),
                    preferred_element_type=jnp.float32,
                ) * recip
                ctx_ref[brows, h * Dh:(h + 1) * Dh] = ctx.astype(jnp.bfloat16)
            acc_ref[brows, :] = lax.dot_general(
                ctx_ref[brows, :], wob, (((1,), (0,)), ((), ())),
                preferred_element_type=jnp.float32,
            ).astype(jnp.bfloat16)
            for c in range(b * chunks_per_b, (b + 1) * chunks_per_b):
                rdmas[(c, 0)] = make_rdma(c, 0)
                rdmas[(c, 0)].start()

        for r in range(LOG_N):
            for c in range(N_CHUNKS):
                rdmas[(c, r)].wait()
                rows = slice(c * crows, (c + 1) * crows)
                acc_ref[rows, :] = acc_ref[rows, :] + cbuf_ref[r, rows, :]
                if r + 1 < LOG_N:
                    rdmas[(c, r + 1)] = make_rdma(c, r + 1)
                    rdmas[(c, r + 1)].start()

        out_ref[...] = acc_ref[...].astype(jnp.float32)

    out2d = pl.pallas_call(
        body,
        out_shape=jax.ShapeDtypeStruct((rows_total, Dm), jnp.float32),
        in_specs=[
            pl.BlockSpec(memory_space=pltpu.VMEM),
            pl.BlockSpec(memory_space=pltpu.VMEM),
            pl.BlockSpec(memory_space=pl.ANY),
            pl.BlockSpec(memory_space=pl.ANY),
            pl.BlockSpec(memory_space=pltpu.VMEM),
        ],
        out_specs=pl.BlockSpec(memory_space=pltpu.VMEM),
        scratch_shapes=[
            pltpu.VMEM((rows_total, HL * Dh), jnp.bfloat16),
            pltpu.VMEM((rows_total, Dm), jnp.bfloat16),
            pltpu.VMEM((LOG_N, rows_total, Dm), jnp.bfloat16),
            pltpu.VMEM((B, Skv, HL, Dh), jnp.float32),
            pltpu.VMEM((B, Skv, HL, Dh), jnp.float32),
            pltpu.SemaphoreType.DMA((N_CHUNKS, LOG_N)),
            pltpu.SemaphoreType.DMA((N_CHUNKS, LOG_N)),
            pltpu.SemaphoreType.DMA((2,)),
        ],
        compiler_params=pltpu.CompilerParams(collective_id=0),
    )(xb, Wq, K_ext, V_ext, Wo)
    return out2d.reshape(B, Sq, Dm)


# device time: 17037 ns/iter; 1.4243x vs baseline; 1.4243x over previous
import jax
import jax.numpy as jnp
from jax import lax
from jax.experimental import pallas as pl
from jax.experimental.pallas import tpu as pltpu

N_DEV = 8
LOG_N = 3
BLK = 64


def kernel(x, Wq, K_ext, V_ext, Wo):
    B, Sq, Dm = x.shape
    _, Dq = Wq.shape
    _, Skv, Hq, Dh = K_ext.shape
    HL = Dq // Dh
    my = lax.axis_index("i")

    xb = x.reshape(B * Sq, Dm)
    K_loc = lax.dynamic_slice_in_dim(K_ext, my * HL, HL, axis=2)
    V_loc = lax.dynamic_slice_in_dim(V_ext, my * HL, HL, axis=2)

    def body(x_ref, wq_ref, k_ref, v_ref, wo_ref, out_ref,
             ctx_ref, acc_ref, cbuf_ref, send_sems, recv_sems):
        my_pos = lax.axis_index("i")

        xors = (1, 3, 4)

        barrier = pltpu.get_barrier_semaphore()
        for r in range(LOG_N):
            pl.semaphore_signal(
                barrier, inc=1,
                device_id=(my_pos ^ xors[r],),
                device_id_type=pl.DeviceIdType.MESH,
            )
        pl.semaphore_wait(barrier, LOG_N)

        q = (lax.dot_general(
            x_ref[...].astype(jnp.bfloat16),
            wq_ref[...].astype(jnp.bfloat16), (((1,), (0,)), ((), ())),
            preferred_element_type=jnp.float32,
        ) * 0.125).astype(jnp.bfloat16)

        i_idx = lax.broadcasted_iota(jnp.int32, (Sq, Skv), 0)
        j_idx = lax.broadcasted_iota(jnp.int32, (Sq, Skv), 1)
        bias = jnp.where((j_idx // BLK) <= (i_idx // BLK), 0.0, -1e9)

        n_chunks = 2 * B
        crows = (B * Sq) // n_chunks

        def make_rdma(c, r):
            rows = pl.ds(c * crows, crows)
            return pltpu.make_async_remote_copy(
                src_ref=acc_ref.at[rows],
                dst_ref=cbuf_ref.at[r, rows],
                send_sem=send_sems.at[c, r],
                recv_sem=recv_sems.at[c, r],
                device_id=(my_pos ^ xors[r],),
                device_id_type=pl.DeviceIdType.MESH,
            )

        rdmas = {}
        for b in range(B):
            for h in range(HL):
                q_bh = q[b * Sq:(b + 1) * Sq, h * Dh:(h + 1) * Dh]
                k_bh = k_ref[b, :, h, :].astype(jnp.bfloat16)
                v_bh = v_ref[b, :, h, :].astype(jnp.bfloat16)
                s = lax.dot_general(
                    q_bh, k_bh, (((1,), (1,)), ((), ())),
                    preferred_element_type=jnp.float32,
                ) + bias
                w = jnp.exp(s)
                recip = 1.0 / jnp.sum(w, axis=-1, keepdims=True)
                ctx = lax.dot_general(
                    w.astype(jnp.bfloat16), v_bh, (((1,), (0,)), ((), ())),
                    preferred_element_type=jnp.float32,
                ) * recip
                ctx_ref[b * Sq:(b + 1) * Sq, h * Dh:(h + 1) * Dh] = (
                    ctx.astype(jnp.bfloat16))
            acc_ref[b * Sq:(b + 1) * Sq, :] = lax.dot_general(
                ctx_ref[b * Sq:(b + 1) * Sq, :],
                wo_ref[...].astype(jnp.bfloat16),
                (((1,), (0,)), ((), ())),
                preferred_element_type=jnp.float32,
            ).astype(jnp.bfloat16)
            for c in range(2 * b, 2 * b + 2):
                rdmas[(c, 0)] = make_rdma(c, 0)
                rdmas[(c, 0)].start()

        for r in range(LOG_N):
            for c in range(n_chunks):
                rdmas[(c, r)].wait()
                rows = slice(c * crows, (c + 1) * crows)
                acc_ref[rows, :] = acc_ref[rows, :] + cbuf_ref[r, rows, :]
                if r + 1 < LOG_N:
                    rdmas[(c, r + 1)] = make_rdma(c, r + 1)
                    rdmas[(c, r + 1)].start()

        out_ref[...] = acc_ref[...].astype(jnp.float32)

    out2d = pl.pallas_call(
        body,
        out_shape=jax.ShapeDtypeStruct((B * Sq, Dm), jnp.float32),
        in_specs=[pl.BlockSpec(memory_space=pltpu.VMEM)] * 5,
        out_specs=pl.BlockSpec(memory_space=pltpu.VMEM),
        scratch_shapes=[
            pltpu.VMEM((B * Sq, HL * Dh), jnp.bfloat16),
            pltpu.VMEM((B * Sq, Dm), jnp.bfloat16),
            pltpu.VMEM((LOG_N, B * Sq, Dm), jnp.bfloat16),
            pltpu.SemaphoreType.DMA((2 * B, LOG_N)),
            pltpu.SemaphoreType.DMA((2 * B, LOG_N)),
        ],
        compiler_params=pltpu.CompilerParams(collective_id=0),
    )(xb, Wq, K_loc, V_loc, Wo)
    return out2d.reshape(B, Sq, Dm)


# device time: 16534 ns/iter; 1.4676x vs baseline; 1.0304x over previous
import jax
import jax.numpy as jnp
from jax import lax
from jax.experimental import pallas as pl
from jax.experimental.pallas import tpu as pltpu

N_DEV = 8
LOG_N = 3
BLK = 64


def kernel(x, Wq, K_ext, V_ext, Wo):
    B, Sq, Dm = x.shape
    _, Dq = Wq.shape
    _, Skv, Hq, Dh = K_ext.shape
    HL = Dq // Dh
    my = lax.axis_index("i")

    xb = x.reshape(B * Sq, Dm)
    K_loc = lax.dynamic_slice_in_dim(K_ext, my * HL, HL, axis=2)
    V_loc = lax.dynamic_slice_in_dim(V_ext, my * HL, HL, axis=2)

    def body(x_ref, wq_ref, k_ref, v_ref, wo_ref, out_ref,
             ctx_ref, acc_ref, cbuf_ref, send_sems, recv_sems):
        my_pos = lax.axis_index("i")

        xors = (1, 3, 4)

        barrier = pltpu.get_barrier_semaphore()
        for r in range(LOG_N):
            pl.semaphore_signal(
                barrier, inc=1,
                device_id=(my_pos ^ xors[r],),
                device_id_type=pl.DeviceIdType.MESH,
            )
        pl.semaphore_wait(barrier, LOG_N)

        q = (lax.dot_general(
            x_ref[...].astype(jnp.bfloat16),
            wq_ref[...].astype(jnp.bfloat16), (((1,), (0,)), ((), ())),
            preferred_element_type=jnp.float32,
        ) * 0.125).astype(jnp.bfloat16)

        i_idx = lax.broadcasted_iota(jnp.int32, (Sq, Skv), 0)
        j_idx = lax.broadcasted_iota(jnp.int32, (Sq, Skv), 1)
        bias = jnp.where((j_idx // BLK) <= (i_idx // BLK), 0.0, -1e9)

        n_chunks = 2 * B
        crows = (B * Sq) // n_chunks
        orders = ((1, 3, 4), (3, 4, 1), (4, 1, 3), (1, 4, 3))

        def make_rdma(c, r):
            rows = pl.ds(c * crows, crows)
            return pltpu.make_async_remote_copy(
                src_ref=acc_ref.at[rows],
                dst_ref=cbuf_ref.at[r, rows],
                send_sem=send_sems.at[c, r],
                recv_sem=recv_sems.at[c, r],
                device_id=(my_pos ^ orders[c][r],),
                device_id_type=pl.DeviceIdType.MESH,
            )

        rdmas = {}
        for b in range(B):
            for h in range(HL):
                q_bh = q[b * Sq:(b + 1) * Sq, h * Dh:(h + 1) * Dh]
                k_bh = k_ref[b, :, h, :].astype(jnp.bfloat16)
                v_bh = v_ref[b, :, h, :].astype(jnp.bfloat16)
                s = lax.dot_general(
                    q_bh, k_bh, (((1,), (1,)), ((), ())),
                    preferred_element_type=jnp.float32,
                ) + bias
                w = jnp.exp(s)
                recip = 1.0 / jnp.sum(w, axis=-1, keepdims=True)
                ctx = lax.dot_general(
                    w.astype(jnp.bfloat16), v_bh, (((1,), (0,)), ((), ())),
                    preferred_element_type=jnp.float32,
                ) * recip
                ctx_ref[b * Sq:(b + 1) * Sq, h * Dh:(h + 1) * Dh] = (
                    ctx.astype(jnp.bfloat16))
            acc_ref[b * Sq:(b + 1) * Sq, :] = lax.dot_general(
                ctx_ref[b * Sq:(b + 1) * Sq, :],
                wo_ref[...].astype(jnp.bfloat16),
                (((1,), (0,)), ((), ())),
                preferred_element_type=jnp.float32,
            ).astype(jnp.bfloat16)
            for c in range(2 * b, 2 * b + 2):
                rdmas[(c, 0)] = make_rdma(c, 0)
                rdmas[(c, 0)].start()

        for r in range(LOG_N):
            for c in range(n_chunks):
                rdmas[(c, r)].wait()
                rows = slice(c * crows, (c + 1) * crows)
                acc_ref[rows, :] = acc_ref[rows, :] + cbuf_ref[r, rows, :]
                if r + 1 < LOG_N:
                    rdmas[(c, r + 1)] = make_rdma(c, r + 1)
                    rdmas[(c, r + 1)].start()

        out_ref[...] = acc_ref[...].astype(jnp.float32)

    out2d = pl.pallas_call(
        body,
        out_shape=jax.ShapeDtypeStruct((B * Sq, Dm), jnp.float32),
        in_specs=[pl.BlockSpec(memory_space=pltpu.VMEM)] * 5,
        out_specs=pl.BlockSpec(memory_space=pltpu.VMEM),
        scratch_shapes=[
            pltpu.VMEM((B * Sq, HL * Dh), jnp.bfloat16),
            pltpu.VMEM((B * Sq, Dm), jnp.bfloat16),
            pltpu.VMEM((LOG_N, B * Sq, Dm), jnp.bfloat16),
            pltpu.SemaphoreType.DMA((2 * B, LOG_N)),
            pltpu.SemaphoreType.DMA((2 * B, LOG_N)),
        ],
        compiler_params=pltpu.CompilerParams(collective_id=0),
    )(xb, Wq, K_loc, V_loc, Wo)
    return out2d.reshape(B, Sq, Dm)
